# baseline (speedup 1.0000x reference)
import functools

import jax
import jax.numpy as jnp
from jax import lax
from jax.experimental import pallas as pl
from jax.experimental.pallas import tpu as pltpu
from jax.experimental.pallas import tpu_sc as plsc

_N_EMB = 1024
_DIM = 64
_NB = 16
_ROWS = _NB * 1024
_BLK = 512
_GRID = _ROWS // _BLK
_CCHUNK = 128
_NCC = _N_EMB // _CCHUNK
_LOSS_SCALE = 2.0 / (_ROWS * _DIM)


def _tc_body(xt_ref, e_ref, x2_ref, e2_ref, idx_ref, loss_ref, m_ref, acc_ref):
    i = pl.program_id(0)
    xt = xt_ref[0]
    e = e_ref[...]
    m_ref[...] = lax.dot_general(e + e, xt, (((1,), (0,)), ((), ())),
                                 preferred_element_type=jnp.float32)
    x2 = x2_ref[...]
    srow = lax.broadcasted_iota(jnp.int32, (_CCHUNK, _BLK), 0)
    best = None
    for c in range(_NCC):
        sl = pl.ds(c * _CCHUNK, _CCHUNK)
        dc = (x2 + e2_ref[sl, :]) - m_ref[sl, :]
        codec = srow + (c * _CCHUNK)
        if best is None:
            best, bidx = dc, codec
        else:
            lt = dc < best
            best = jnp.where(lt, dc, best)
            bidx = jnp.where(lt, codec, bidx)
    rowmin = jnp.min(best, axis=0, keepdims=True)
    idx = jnp.min(jnp.where(best == rowmin, bidx, _N_EMB), axis=0)
    idx_ref[0, 0, :] = idx

    @pl.when(i == 0)
    def _():
        acc_ref[0] = 0.0

    acc_ref[0] += jnp.sum(rowmin)

    @pl.when(i == _GRID - 1)
    def _():
        loss_ref[0, 0] = acc_ref[0] * _LOSS_SCALE


def _tc_call(xt, embedding, x2, e2, interpret=False):
    nsub = 1024 // _BLK
    return pl.pallas_call(
        _tc_body,
        grid=(_GRID,),
        in_specs=[
            pl.BlockSpec((1, _DIM, _BLK), lambda i: (i // nsub, 0, i % nsub)),
            pl.BlockSpec((_N_EMB, _DIM), lambda i: (0, 0)),
            pl.BlockSpec((1, _BLK), lambda i: (0, i)),
            pl.BlockSpec((_N_EMB, 1), lambda i: (0, 0)),
        ],
        out_specs=[
            pl.BlockSpec((1, 1, _BLK), lambda i: (i, 0, 0)),
            pl.BlockSpec(memory_space=pltpu.SMEM, block_shape=(1, 1),
                         index_map=lambda i: (0, 0)),
        ],
        out_shape=[
            jax.ShapeDtypeStruct((_GRID, 1, _BLK), jnp.int32),
            jax.ShapeDtypeStruct((1, 1), jnp.float32),
        ],
        scratch_shapes=[pltpu.VMEM((_N_EMB, _BLK), jnp.float32),
                        pltpu.SMEM((1,), jnp.float32)],
        interpret=interpret,
    )(xt, embedding, x2, e2)


_NW = 32
_BPW = _ROWS // _NW
_CHUNK = 128
_NCHUNK = _BPW // _CHUNK


def _sc_gather(embedding, idx):
    mesh = plsc.VectorSubcoreMesh(core_axis_name="c", subcore_axis_name="s")

    @functools.partial(
        pl.kernel,
        mesh=mesh,
        out_type=jax.ShapeDtypeStruct((_NB, 1024, _DIM), jnp.float32),
        compiler_params=pltpu.CompilerParams(use_tc_tiling_on_sc=False),
        scratch_types=[
            pltpu.VMEM((_NCHUNK, _CHUNK), jnp.int32),
            pltpu.VMEM((_BPW, _DIM), jnp.float32),
            pltpu.SemaphoreType.DMA,
        ],
    )
    def k(table_hbm, idx_hbm, out_hbm, idx_v, rows_v, sem):
        wid = lax.axis_index("s") * 2 + lax.axis_index("c")
        base = wid * _BPW
        for c in range(_NCHUNK):
            pltpu.sync_copy(idx_hbm.at[pl.ds(base + c * _CHUNK, _CHUNK)],
                            idx_v.at[c])
        copies = []
        for c in range(_NCHUNK):
            copies.append(pltpu.async_copy(
                table_hbm.at[idx_v.at[c]],
                rows_v.at[pl.ds(c * _CHUNK, _CHUNK)], sem))
        for cp in copies:
            cp.wait()
        b = wid // (1024 // _BPW)
        r0 = (wid % (1024 // _BPW)) * _BPW
        pltpu.sync_copy(rows_v, out_hbm.at[b, pl.ds(r0, _BPW), :])

    return k(embedding, idx)


def kernel(inputs, embedding):
    xt = inputs.transpose(0, 2, 1)
    x2 = jnp.sum(inputs.reshape(_ROWS, _DIM) ** 2, axis=1).reshape(1, _ROWS)
    e2 = jnp.sum(embedding ** 2, axis=1).reshape(_N_EMB, 1)
    idx2d, loss = _tc_call(xt, embedding, x2, e2)
    idx = idx2d.reshape(_ROWS)
    quantized = _sc_gather(embedding, idx)
    quantized_st = inputs + (quantized - inputs)
    return (quantized_st, loss[0, 0], idx[:, None])


# speedup vs baseline: 1.0989x; 1.0989x over previous
import functools

import jax
import jax.numpy as jnp
from jax import lax
from jax.experimental import pallas as pl
from jax.experimental.pallas import tpu as pltpu
from jax.experimental.pallas import tpu_sc as plsc

_N_EMB = 1024
_DIM = 64
_NB = 16
_ROWS = _NB * 1024
_NBH = _NB // 2
_ROWS_H = _ROWS // 2
_BLK = 512
_GRID_H = _ROWS_H // _BLK
_CCHUNK = 128
_NCC = _N_EMB // _CCHUNK
_LOSS_SCALE = 2.0 / (_ROWS * _DIM)


def _tc_body(xt_ref, e_ref, x2_ref, e2_ref, idx_ref, loss_ref, m_ref, acc_ref):
    i = pl.program_id(0)
    xt = xt_ref[0]
    e = e_ref[...]
    m_ref[...] = lax.dot_general(e + e, xt, (((1,), (0,)), ((), ())),
                                 preferred_element_type=jnp.float32)
    x2 = x2_ref[...]
    srow = lax.broadcasted_iota(jnp.int32, (_CCHUNK, _BLK), 0)
    best = None
    for c in range(_NCC):
        sl = pl.ds(c * _CCHUNK, _CCHUNK)
        dc = (x2 + e2_ref[sl, :]) - m_ref[sl, :]
        codec = srow + (c * _CCHUNK)
        if best is None:
            best, bidx = dc, codec
        else:
            lt = dc < best
            best = jnp.where(lt, dc, best)
            bidx = jnp.where(lt, codec, bidx)
    rowmin = jnp.min(best, axis=0, keepdims=True)
    idx = jnp.min(jnp.where(best == rowmin, bidx, _N_EMB), axis=0)
    idx_ref[0, 0, :] = idx

    @pl.when(i == 0)
    def _():
        acc_ref[0] = 0.0

    acc_ref[0] += jnp.sum(rowmin)

    @pl.when(i == _GRID_H - 1)
    def _():
        loss_ref[0, 0] = acc_ref[0] * _LOSS_SCALE


def _tc_call(xt, embedding, x2, e2, half, interpret=False):
    nsub = 1024 // _BLK
    boff = half * _NBH
    xoff = half * (_ROWS_H // _BLK)
    return pl.pallas_call(
        _tc_body,
        grid=(_GRID_H,),
        in_specs=[
            pl.BlockSpec((1, _DIM, _BLK),
                         lambda i: (boff + i // nsub, 0, i % nsub)),
            pl.BlockSpec((_N_EMB, _DIM), lambda i: (0, 0)),
            pl.BlockSpec((1, _BLK), lambda i: (0, xoff + i)),
            pl.BlockSpec((_N_EMB, 1), lambda i: (0, 0)),
        ],
        out_specs=[
            pl.BlockSpec((1, 1, _BLK), lambda i: (i, 0, 0)),
            pl.BlockSpec(memory_space=pltpu.SMEM, block_shape=(1, 1),
                         index_map=lambda i: (0, 0)),
        ],
        out_shape=[
            jax.ShapeDtypeStruct((_GRID_H, 1, _BLK), jnp.int32),
            jax.ShapeDtypeStruct((1, 1), jnp.float32),
        ],
        scratch_shapes=[pltpu.VMEM((_N_EMB, _BLK), jnp.float32),
                        pltpu.SMEM((1,), jnp.float32)],
        interpret=interpret,
    )(xt, embedding, x2, e2)


_NW = 32
_BPW = _ROWS_H // _NW
_CHUNK = 128
_NCHUNK = _BPW // _CHUNK


def _sc_gather_half(embedding, idx):
    mesh = plsc.VectorSubcoreMesh(core_axis_name="c", subcore_axis_name="s")

    @functools.partial(
        pl.kernel,
        mesh=mesh,
        out_type=jax.ShapeDtypeStruct((_NBH, 1024, _DIM), jnp.float32),
        compiler_params=pltpu.CompilerParams(use_tc_tiling_on_sc=False),
        scratch_types=[
            pltpu.VMEM((_NCHUNK, _CHUNK), jnp.int32),
            pltpu.VMEM((_BPW, _DIM), jnp.float32),
            pltpu.SemaphoreType.DMA,
        ],
    )
    def k(table_hbm, idx_hbm, out_hbm, idx_v, rows_v, sem):
        wid = lax.axis_index("s") * 2 + lax.axis_index("c")
        base = wid * _BPW
        for c in range(_NCHUNK):
            pltpu.sync_copy(idx_hbm.at[pl.ds(base + c * _CHUNK, _CHUNK)],
                            idx_v.at[c])
        copies = []
        for c in range(_NCHUNK):
            copies.append(pltpu.async_copy(
                table_hbm.at[idx_v.at[c]],
                rows_v.at[pl.ds(c * _CHUNK, _CHUNK)], sem))
        for cp in copies:
            cp.wait()
        b = base // 1024
        r0 = base % 1024
        pltpu.sync_copy(rows_v, out_hbm.at[b, pl.ds(r0, _BPW), :])

    return k(embedding, idx)


def kernel(inputs, embedding):
    xt = inputs.transpose(0, 2, 1)
    x2 = jnp.sum(inputs.reshape(_ROWS, _DIM) ** 2, axis=1).reshape(1, _ROWS)
    e2 = jnp.sum(embedding ** 2, axis=1).reshape(_N_EMB, 1)
    idx_a, loss_a = _tc_call(xt, embedding, x2, e2, 0)
    q_a = _sc_gather_half(embedding, idx_a.reshape(_ROWS_H))
    idx_b, loss_b = _tc_call(xt, embedding, x2, e2, 1)
    q_b = _sc_gather_half(embedding, idx_b.reshape(_ROWS_H))
    quantized = jnp.concatenate([q_a, q_b], axis=0)
    idx = jnp.concatenate([idx_a.reshape(_ROWS_H), idx_b.reshape(_ROWS_H)])
    loss = loss_a[0, 0] + loss_b[0, 0]
    return (quantized, loss, idx[:, None])


# speedup vs baseline: 1.1354x; 1.0332x over previous
import functools

import jax
import jax.numpy as jnp
from jax import lax
from jax.experimental import pallas as pl
from jax.experimental.pallas import tpu as pltpu
from jax.experimental.pallas import tpu_sc as plsc

_N_EMB = 1024
_DIM = 64
_NB = 16
_ROWS = _NB * 1024
_NBH = _NB // 2
_ROWS_H = _ROWS // 2
_BLK = 512
_GRID_H = _ROWS_H // _BLK
_CCHUNK = 128
_NCC = _N_EMB // _CCHUNK
_LOSS_SCALE = 2.0 / (_ROWS * _DIM)


def _tc_body(xt_ref, e_ref, x2_ref, e2_ref, idx_ref, loss_ref, m_ref, acc_ref):
    i = pl.program_id(0)
    xt = xt_ref[0]
    e = e_ref[...]
    m_ref[...] = lax.dot_general(e + e, xt, (((1,), (0,)), ((), ())),
                                 preferred_element_type=jnp.float32)
    x2 = x2_ref[...]
    srow = lax.broadcasted_iota(jnp.int32, (_CCHUNK, _BLK), 0)
    best = None
    for c in range(_NCC):
        sl = pl.ds(c * _CCHUNK, _CCHUNK)
        dc = (x2 + e2_ref[sl, :]) - m_ref[sl, :]
        codec = srow + (c * _CCHUNK)
        if best is None:
            best, bidx = dc, codec
        else:
            lt = dc < best
            best = jnp.where(lt, dc, best)
            bidx = jnp.where(lt, codec, bidx)
    rowmin = jnp.min(best, axis=0, keepdims=True)
    idx = jnp.min(jnp.where(best == rowmin, bidx, _N_EMB), axis=0)
    idx_ref[0, 0, :] = idx

    @pl.when(i == 0)
    def _():
        acc_ref[0] = 0.0

    acc_ref[0] += jnp.sum(rowmin)

    @pl.when(i == _GRID_H - 1)
    def _():
        loss_ref[0, 0] = acc_ref[0] * _LOSS_SCALE


def _tc_call(xt, embedding, x2, e2, half, interpret=False):
    nsub = 1024 // _BLK
    boff = half * _NBH
    xoff = half * (_ROWS_H // _BLK)
    return pl.pallas_call(
        _tc_body,
        grid=(_GRID_H,),
        in_specs=[
            pl.BlockSpec((1, _DIM, _BLK),
                         lambda i: (boff + i // nsub, 0, i % nsub)),
            pl.BlockSpec((_N_EMB, _DIM), lambda i: (0, 0)),
            pl.BlockSpec((1, _BLK), lambda i: (0, xoff + i)),
            pl.BlockSpec((_N_EMB, 1), lambda i: (0, 0)),
        ],
        out_specs=[
            pl.BlockSpec((1, 1, _BLK), lambda i: (i, 0, 0)),
            pl.BlockSpec(memory_space=pltpu.SMEM, block_shape=(1, 1),
                         index_map=lambda i: (0, 0)),
        ],
        out_shape=[
            jax.ShapeDtypeStruct((_GRID_H, 1, _BLK), jnp.int32),
            jax.ShapeDtypeStruct((1, 1), jnp.float32),
        ],
        scratch_shapes=[pltpu.VMEM((_N_EMB, _BLK), jnp.float32),
                        pltpu.SMEM((1,), jnp.float32)],
        interpret=interpret,
    )(xt, embedding, x2, e2)


def _tr_body(i_ref, o_ref):
    v = i_ref[0]
    o_ref[0, :, :512] = v[:, :_DIM].T
    o_ref[0, :, 512:] = v[:, _DIM:].T


def _tr_call(q, interpret=False):
    return pl.pallas_call(
        _tr_body,
        grid=(_NB,),
        in_specs=[pl.BlockSpec((1, 512, 128), lambda i: (i, 0, 0))],
        out_specs=pl.BlockSpec((1, _DIM, 1024), lambda i: (i, 0, 0)),
        out_shape=jax.ShapeDtypeStruct((_NB, _DIM, 1024), jnp.float32),
        interpret=interpret,
    )(q)


_NW = 32
_BPW = _ROWS_H // _NW
_CHUNK = 128
_NCHUNK = _BPW // _CHUNK


def _sc_gather_half(embedding, idx):
    mesh = plsc.VectorSubcoreMesh(core_axis_name="c", subcore_axis_name="s")

    @functools.partial(
        pl.kernel,
        mesh=mesh,
        out_type=jax.ShapeDtypeStruct((_NBH, 512, 2 * _DIM), jnp.float32),
        compiler_params=pltpu.CompilerParams(use_tc_tiling_on_sc=False),
        scratch_types=[
            pltpu.VMEM((_NCHUNK, _CHUNK), jnp.int32),
            pltpu.VMEM((_BPW, _DIM), jnp.float32),
            pltpu.SemaphoreType.DMA,
        ],
    )
    def k(table_hbm, idx_hbm, out_hbm, idx_v, rows_v, sem):
        wid = lax.axis_index("s") * 2 + lax.axis_index("c")
        base = wid * _BPW
        for c in range(_NCHUNK):
            pltpu.sync_copy(idx_hbm.at[pl.ds(base + c * _CHUNK, _CHUNK)],
                            idx_v.at[c])
        copies = []
        for c in range(_NCHUNK):
            copies.append(pltpu.async_copy(
                table_hbm.at[idx_v.at[c]],
                rows_v.at[pl.ds(c * _CHUNK, _CHUNK)], sem))
        for cp in copies:
            cp.wait()
        b = base // 1024
        p = (base % 1024) // 512
        k0 = base % 512
        pltpu.sync_copy(rows_v,
                        out_hbm.at[b, pl.ds(k0, _BPW), pl.ds(_DIM * p, _DIM)])

    return k(embedding, idx)


def kernel(inputs, embedding):
    xt = inputs.transpose(0, 2, 1)
    x2 = jnp.sum(inputs.reshape(_ROWS, _DIM) ** 2, axis=1).reshape(1, _ROWS)
    e2 = jnp.sum(embedding ** 2, axis=1).reshape(_N_EMB, 1)
    idx_a, loss_a = _tc_call(xt, embedding, x2, e2, 0)
    q_a = _sc_gather_half(embedding, idx_a.reshape(_ROWS_H))
    idx_b, loss_b = _tc_call(xt, embedding, x2, e2, 1)
    q_b = _sc_gather_half(embedding, idx_b.reshape(_ROWS_H))
    quantized = jnp.concatenate([q_a, q_b], axis=0)
    qt = _tr_call(quantized)
    idx = jnp.concatenate([idx_a.reshape(_ROWS_H), idx_b.reshape(_ROWS_H)])
    loss = loss_a[0, 0] + loss_b[0, 0]
    return (qt.transpose(0, 2, 1), loss, idx[:, None])


# speedup vs baseline: 1.2015x; 1.0582x over previous
import functools

import jax
import jax.numpy as jnp
from jax import lax
from jax.experimental import pallas as pl
from jax.experimental.pallas import tpu as pltpu
from jax.experimental.pallas import tpu_sc as plsc

_N_EMB = 1024
_DIM = 64
_NB = 16
_ROWS = _NB * 1024
_NBH = _NB // 2
_ROWS_H = _ROWS // 2
_BLK = 512
_GRID_H = _ROWS_H // _BLK
_CCHUNK = 128
_NCC = _N_EMB // _CCHUNK
_LOSS_SCALE = 2.0 / (_ROWS * _DIM)


def _tc_body(xt_ref, e_ref, x2_ref, e2_ref, idx_ref, loss_ref, m_ref, acc_ref):
    i = pl.program_id(0)
    xt = xt_ref[0]
    e = e_ref[...]
    m_ref[...] = lax.dot_general(e + e, xt, (((1,), (0,)), ((), ())),
                                 preferred_element_type=jnp.float32)
    x2 = x2_ref[...]
    srow = lax.broadcasted_iota(jnp.int32, (_CCHUNK, _BLK), 0)
    best = None
    for c in range(_NCC):
        sl = pl.ds(c * _CCHUNK, _CCHUNK)
        dc = (x2 + e2_ref[sl, :]) - m_ref[sl, :]
        codec = srow + (c * _CCHUNK)
        if best is None:
            best, bidx = dc, codec
        else:
            lt = dc < best
            best = jnp.where(lt, dc, best)
            bidx = jnp.where(lt, codec, bidx)
    rowmin = jnp.min(best, axis=0, keepdims=True)
    idx = jnp.min(jnp.where(best == rowmin, bidx, _N_EMB), axis=0)
    idx_ref[0, 0, :] = idx

    @pl.when(i == 0)
    def _():
        acc_ref[0] = 0.0

    acc_ref[0] += jnp.sum(rowmin)

    @pl.when(i == pl.num_programs(0) - 1)
    def _():
        loss_ref[0, 0] = acc_ref[0] * _LOSS_SCALE


def _tc_call(xt, embedding, x2, e2, half=0, nsteps=_ROWS // _BLK,
             interpret=False):
    nsub = 1024 // _BLK
    boff = half * _NBH
    xoff = half * (_ROWS_H // _BLK)
    return pl.pallas_call(
        _tc_body,
        grid=(nsteps,),
        in_specs=[
            pl.BlockSpec((1, _DIM, _BLK),
                         lambda i: (boff + i // nsub, 0, i % nsub)),
            pl.BlockSpec((_N_EMB, _DIM), lambda i: (0, 0)),
            pl.BlockSpec((1, _BLK), lambda i: (0, xoff + i)),
            pl.BlockSpec((_N_EMB, 1), lambda i: (0, 0)),
        ],
        out_specs=[
            pl.BlockSpec((1, 1, _BLK), lambda i: (i, 0, 0)),
            pl.BlockSpec(memory_space=pltpu.SMEM, block_shape=(1, 1),
                         index_map=lambda i: (0, 0)),
        ],
        out_shape=[
            jax.ShapeDtypeStruct((nsteps, 1, _BLK), jnp.int32),
            jax.ShapeDtypeStruct((1, 1), jnp.float32),
        ],
        scratch_shapes=[pltpu.VMEM((_N_EMB, _BLK), jnp.float32),
                        pltpu.SMEM((1,), jnp.float32)],
        interpret=interpret,
    )(xt, embedding, x2, e2)


def _tr_body(i_ref, o_ref):
    v = i_ref[0]
    o_ref[0, :, :512] = v[:, :_DIM].T
    o_ref[0, :, 512:] = v[:, _DIM:].T


def _tr_call(q, interpret=False):
    return pl.pallas_call(
        _tr_body,
        grid=(_NB,),
        in_specs=[pl.BlockSpec((1, 512, 128), lambda i: (i, 0, 0))],
        out_specs=pl.BlockSpec((1, _DIM, 1024), lambda i: (i, 0, 0)),
        out_shape=jax.ShapeDtypeStruct((_NB, _DIM, 1024), jnp.float32),
        interpret=interpret,
    )(q)


_NW = 32
_BPW = _ROWS // _NW
_CHUNK = 128
_NCHUNK = _BPW // _CHUNK


def _sc_gather_half(embedding, idx):
    mesh = plsc.VectorSubcoreMesh(core_axis_name="c", subcore_axis_name="s")

    @functools.partial(
        pl.kernel,
        mesh=mesh,
        out_type=jax.ShapeDtypeStruct((_NB, 512, 2 * _DIM), jnp.float32),
        compiler_params=pltpu.CompilerParams(use_tc_tiling_on_sc=False),
        scratch_types=[
            pltpu.VMEM((_NCHUNK, _CHUNK), jnp.int32),
            pltpu.VMEM((_BPW, _DIM), jnp.float32),
            pltpu.SemaphoreType.DMA,
        ],
    )
    def k(table_hbm, idx_hbm, out_hbm, idx_v, rows_v, sem):
        wid = lax.axis_index("s") * 2 + lax.axis_index("c")
        base = wid * _BPW
        for c in range(_NCHUNK):
            pltpu.sync_copy(idx_hbm.at[pl.ds(base + c * _CHUNK, _CHUNK)],
                            idx_v.at[c])
        copies = []
        for c in range(_NCHUNK):
            copies.append(pltpu.async_copy(
                table_hbm.at[idx_v.at[c]],
                rows_v.at[pl.ds(c * _CHUNK, _CHUNK)], sem))
        for cp in copies:
            cp.wait()
        b = base // 1024
        p = (base % 1024) // 512
        k0 = base % 512
        pltpu.sync_copy(rows_v,
                        out_hbm.at[b, pl.ds(k0, _BPW), pl.ds(_DIM * p, _DIM)])

    return k(embedding, idx)


def kernel(inputs, embedding):
    xt = inputs.transpose(0, 2, 1)
    x2 = jnp.sum(inputs.reshape(_ROWS, _DIM) ** 2, axis=1).reshape(1, _ROWS)
    e2 = jnp.sum(embedding ** 2, axis=1).reshape(_N_EMB, 1)
    idx2d, loss = _tc_call(xt, embedding, x2, e2)
    idx = idx2d.reshape(_ROWS)
    quantized = _sc_gather_half(embedding, idx)
    qt = _tr_call(quantized)
    return (qt.transpose(0, 2, 1), loss[0, 0], idx[:, None])
